# Initial kernel scaffold; baseline (speedup 1.0000x reference)
import jax
import jax.numpy as jnp
from jax.experimental import pallas as pl


def kernel(x, x_edge, edge_distance, edge_index, wigner_inv, W1, b1, ln_g, ln_b, W2, b2, to_m, out_idx):
    raise NotImplementedError("write your pallas kernel here")


# speedup vs baseline: 8.5154x; 8.5154x over previous
import functools

import numpy as np
import jax
import jax.numpy as jnp
from jax import lax
from jax.experimental import pallas as pl
from jax.experimental.pallas import tpu as pltpu
from jax.experimental.pallas import tpu_sc as plsc

N = 10000
E = 160000
FULL = 16
C = 64
CUTOFF = 12.0
RESCALE = 23.395238876342773
COLS = (0, 2, 6, 12)

B_EDGE = 640
BATCH = 80
PER_TILE = E // 16
N_PER_TILE = N // 16


def _emb_body(xe_ref, dist_ref, wig_ref, w1_ref, b1_ref, g_ref, bb_ref,
              w2_ref, b2_ref, p_ref, s_ref, out_ref):
    xe = xe_ref[...]
    h = jnp.dot(xe, w1_ref[...], preferred_element_type=jnp.float32) + b1_ref[...]
    mu = jnp.mean(h, axis=-1, keepdims=True)
    var = jnp.mean((h - mu) ** 2, axis=-1, keepdims=True)
    h = (h - mu) * lax.rsqrt(var + 1e-5) * g_ref[...] + bb_ref[...]
    h = h * (1.0 / (1.0 + jnp.exp(-h)))
    rad = jnp.dot(h, w2_ref[...], preferred_element_type=jnp.float32) + b2_ref[...]
    d = dist_ref[...] * (1.0 / CUTOFF)
    d5 = (d * d) * (d * d) * d
    env = 1.0 + d5 * (-21.0 + d * (35.0 - 15.0 * d))
    env = jnp.where(d < 1.0, env, 0.0) * (1.0 / RESCALE)
    rad = rad * env
    w4 = jnp.dot(wig_ref[...], p_ref[...], preferred_element_type=jnp.float32)
    s_mat = s_ref[...]
    acc = None
    for l in range(4):
        wl = jnp.dot(w4[:, l * 16:(l + 1) * 16], s_mat,
                     preferred_element_type=jnp.float32)
        rl = rad[:, l * 64:(l + 1) * 64]
        r2 = jnp.concatenate([rl] * 16, axis=1)
        term = wl * r2
        acc = term if acc is None else acc + term
    out_ref[...] = acc


def _compute_emb(x_edge, edge_distance, wigner_inv, W1, b1, ln_g, ln_b, W2, b2):
    grid = E // B_EDGE
    dist2 = edge_distance.reshape(E, 1)
    wigf = wigner_inv.reshape(E, FULL * FULL)
    p = np.zeros((FULL * FULL, 64), dtype=np.float32)
    for l, col in enumerate(COLS):
        for i in range(FULL):
            p[i * FULL + col, l * 16 + i] = 1.0
    s = np.zeros((16, 1024), dtype=np.float32)
    for f in range(1024):
        s[f // 64, f] = 1.0
    const = lambda shape: pl.BlockSpec(shape, lambda i: (0, 0))
    return pl.pallas_call(
        _emb_body,
        grid=(grid,),
        in_specs=[
            pl.BlockSpec((B_EDGE, 64), lambda i: (i, 0)),
            pl.BlockSpec((B_EDGE, 1), lambda i: (i, 0)),
            pl.BlockSpec((B_EDGE, FULL * FULL), lambda i: (i, 0)),
            const((64, 64)),
            const((1, 64)),
            const((1, 64)),
            const((1, 64)),
            const((64, 256)),
            const((1, 256)),
            const((FULL * FULL, 64)),
            const((16, 1024)),
        ],
        out_specs=pl.BlockSpec((B_EDGE, 1024), lambda i: (i, 0)),
        out_shape=jax.ShapeDtypeStruct((E, 1024), jnp.float32),
    )(x_edge, dist2, wigf, W1, b1.reshape(1, 64), ln_g.reshape(1, 64),
      ln_b.reshape(1, 64), W2, b2.reshape(1, 256), jnp.asarray(p), jnp.asarray(s))


def _sc_scatter(emb8, dst, x8):
    mesh = plsc.VectorSubcoreMesh(core_axis_name="c", subcore_axis_name="s")

    @functools.partial(
        pl.kernel,
        mesh=mesh,
        out_type=jax.ShapeDtypeStruct((N, 8, 128), jnp.float32),
        scratch_types=[
            pltpu.VMEM((BATCH,), jnp.int32),
            pltpu.VMEM((BATCH, 128), jnp.float32),
            pltpu.VMEM_SHARED((N, 128), jnp.float32),
        ],
    )
    def scatter_kernel(emb_hbm, dst_hbm, x_hbm, out_hbm, idx_v, rows_v, acc):
        c = lax.axis_index("c")
        s = lax.axis_index("s")
        row_lo = s * N_PER_TILE
        edge_base = s * PER_TILE
        for cl in range(4):
            chunk = c * 4 + cl
            pltpu.sync_copy(x_hbm.at[pl.ds(row_lo, N_PER_TILE), chunk],
                            acc.at[pl.ds(row_lo, N_PER_TILE)])
            plsc.subcore_barrier()

            def body(b, carry):
                lo = edge_base + b * BATCH
                pltpu.sync_copy(dst_hbm.at[pl.ds(lo, BATCH)], idx_v)
                pltpu.sync_copy(emb_hbm.at[pl.ds(lo, BATCH), chunk], rows_v)
                pltpu.sync_copy(rows_v, acc.at[idx_v], add=True)
                return carry

            lax.fori_loop(0, PER_TILE // BATCH, body, 0)
            plsc.subcore_barrier()
            pltpu.sync_copy(acc.at[pl.ds(row_lo, N_PER_TILE)],
                            out_hbm.at[pl.ds(row_lo, N_PER_TILE), chunk])
            plsc.subcore_barrier()

    return scatter_kernel(emb8, dst, x8)


def kernel(x, x_edge, edge_distance, edge_index, wigner_inv,
           W1, b1, ln_g, ln_b, W2, b2, to_m, out_idx):
    emb = _compute_emb(x_edge, edge_distance, wigner_inv,
                       W1, b1, ln_g, ln_b, W2, b2)
    dst = edge_index[1]
    out8 = _sc_scatter(emb.reshape(E, 8, 128), dst, x.reshape(N, 8, 128))
    return out8.reshape(N, FULL, C)


# speedup vs baseline: 15.1064x; 1.7740x over previous
import functools

import numpy as np
import jax
import jax.numpy as jnp
from jax import lax
from jax.experimental import pallas as pl
from jax.experimental.pallas import tpu as pltpu
from jax.experimental.pallas import tpu_sc as plsc

N = 10000
E = 160000
FULL = 16
C = 64
CUTOFF = 12.0
RESCALE = 23.395238876342773
COLS = (0, 2, 6, 12)

B_EDGE = 640
BATCH = 128
PER_TILE = E // 16
NB_FULL = PER_TILE // BATCH
TAIL = PER_TILE - NB_FULL * BATCH
ROWS_T = 624


def _emb_body(xe_ref, dist_ref, wig_ref, w1_ref, b1_ref, g_ref, bb_ref,
              w2_ref, b2_ref, p_ref, s_ref, out_ref):
    xe = xe_ref[...]
    h = jnp.dot(xe, w1_ref[...], preferred_element_type=jnp.float32) + b1_ref[...]
    mu = jnp.mean(h, axis=-1, keepdims=True)
    var = jnp.mean((h - mu) ** 2, axis=-1, keepdims=True)
    h = (h - mu) * lax.rsqrt(var + 1e-5) * g_ref[...] + bb_ref[...]
    h = h * (1.0 / (1.0 + jnp.exp(-h)))
    rad = jnp.dot(h, w2_ref[...], preferred_element_type=jnp.float32) + b2_ref[...]
    d = dist_ref[...] * (1.0 / CUTOFF)
    d5 = (d * d) * (d * d) * d
    env = 1.0 + d5 * (-21.0 + d * (35.0 - 15.0 * d))
    env = jnp.where(d < 1.0, env, 0.0) * (1.0 / RESCALE)
    rad = rad * env
    w4 = jnp.dot(wig_ref[...], p_ref[...], preferred_element_type=jnp.float32)
    s_mat = s_ref[...]
    acc = None
    for l in range(4):
        wl = jnp.dot(w4[:, l * 16:(l + 1) * 16], s_mat,
                     preferred_element_type=jnp.float32)
        rl = rad[:, l * 64:(l + 1) * 64]
        r2 = jnp.concatenate([rl] * 16, axis=1)
        term = wl * r2
        acc = term if acc is None else acc + term
    out_ref[...] = acc


def _compute_emb(x_edge, edge_distance, wigner_inv, W1, b1, ln_g, ln_b, W2, b2):
    grid = E // B_EDGE
    dist2 = edge_distance.reshape(E, 1)
    wigf = wigner_inv.reshape(E, FULL * FULL)
    p = np.zeros((FULL * FULL, 64), dtype=np.float32)
    for l, col in enumerate(COLS):
        for i in range(FULL):
            p[i * FULL + col, l * 16 + i] = 1.0
    s = np.zeros((16, 1024), dtype=np.float32)
    for f in range(1024):
        s[f // 64, f] = 1.0
    const = lambda shape: pl.BlockSpec(shape, lambda i: (0, 0))
    return pl.pallas_call(
        _emb_body,
        grid=(grid,),
        in_specs=[
            pl.BlockSpec((B_EDGE, 64), lambda i: (i, 0)),
            pl.BlockSpec((B_EDGE, 1), lambda i: (i, 0)),
            pl.BlockSpec((B_EDGE, FULL * FULL), lambda i: (i, 0)),
            const((64, 64)),
            const((1, 64)),
            const((1, 64)),
            const((1, 64)),
            const((64, 256)),
            const((1, 256)),
            const((FULL * FULL, 64)),
            const((16, 1024)),
        ],
        out_specs=pl.BlockSpec((B_EDGE, 1024), lambda i: (i, 0)),
        out_shape=jax.ShapeDtypeStruct((E, 1024), jnp.float32),
    )(x_edge, dist2, wigf, W1, b1.reshape(1, 64), ln_g.reshape(1, 64),
      ln_b.reshape(1, 64), W2, b2.reshape(1, 256), jnp.asarray(p), jnp.asarray(s))


DEPTH = 2


def _sc_scatter(emb2, dst, x2):
    mesh = plsc.VectorSubcoreMesh(core_axis_name="c", subcore_axis_name="s")

    @functools.partial(
        pl.kernel,
        mesh=mesh,
        out_type=jax.ShapeDtypeStruct((N, 1024), jnp.float32),
        scratch_types=[
            *[pltpu.VMEM((BATCH,), jnp.int32) for _ in range(DEPTH)],
            *[pltpu.VMEM((BATCH, 128), jnp.float32) for _ in range(DEPTH)],
            pltpu.VMEM((TAIL,), jnp.int32),
            pltpu.VMEM((TAIL, 128), jnp.float32),
            pltpu.VMEM_SHARED((N, 128), jnp.float32),
            *[pltpu.SemaphoreType.DMA for _ in range(2 * DEPTH)],
        ],
    )
    def scatter_kernel(emb_hbm, dst_hbm, x_hbm, out_hbm, *rest):
        idxs = rest[:DEPTH]
        rows = rest[DEPTH:2 * DEPTH]
        idx_t = rest[2 * DEPTH]
        rows_t = rest[2 * DEPTH + 1]
        acc = rest[2 * DEPTH + 2]
        isems = rest[2 * DEPTH + 3:2 * DEPTH + 3 + DEPTH]
        rsems = rest[2 * DEPTH + 3 + DEPTH:]
        c = lax.axis_index("c")
        s = lax.axis_index("s")
        row_lo = s * ROWS_T
        edge_base = s * PER_TILE

        def start_batch(b, j, col):
            lo = edge_base + b * BATCH
            pltpu.async_copy(dst_hbm.at[pl.ds(lo, BATCH)], idxs[j], isems[j])
            pltpu.async_copy(emb_hbm.at[pl.ds(lo, BATCH), pl.ds(col, 128)],
                             rows[j], rsems[j])

        for cl in range(4):
            col = (c * 4 + cl) * 128
            pltpu.sync_copy(x_hbm.at[pl.ds(row_lo, ROWS_T), pl.ds(col, 128)],
                            acc.at[pl.ds(row_lo, ROWS_T)])

            @pl.when(s == 15)
            def _():
                pltpu.sync_copy(
                    x_hbm.at[pl.ds(16 * ROWS_T, N - 16 * ROWS_T), pl.ds(col, 128)],
                    acc.at[pl.ds(16 * ROWS_T, N - 16 * ROWS_T)])

            plsc.subcore_barrier()

            for j in range(DEPTH):
                start_batch(j, j, col)

            def body(k, carry):
                for j in range(DEPTH):
                    b = k * DEPTH + j
                    pltpu.make_async_copy(dst_hbm.at[pl.ds(0, BATCH)],
                                          idxs[j], isems[j]).wait()
                    pltpu.make_async_copy(
                        emb_hbm.at[pl.ds(0, BATCH), pl.ds(col, 128)],
                        rows[j], rsems[j]).wait()
                    pltpu.sync_copy(rows[j], acc.at[idxs[j]], add=True)

                    @pl.when(b + DEPTH < NB_FULL)
                    def _():
                        start_batch(b + DEPTH, j, col)
                return carry

            lax.fori_loop(0, NB_FULL // DEPTH, body, 0)
            lo_t = edge_base + NB_FULL * BATCH
            pltpu.sync_copy(dst_hbm.at[pl.ds(lo_t, TAIL)], idx_t)
            pltpu.sync_copy(emb_hbm.at[pl.ds(lo_t, TAIL), pl.ds(col, 128)],
                            rows_t)
            pltpu.sync_copy(rows_t, acc.at[idx_t], add=True)

            plsc.subcore_barrier()
            pltpu.sync_copy(acc.at[pl.ds(row_lo, ROWS_T)],
                            out_hbm.at[pl.ds(row_lo, ROWS_T), pl.ds(col, 128)])

            @pl.when(s == 15)
            def _():
                pltpu.sync_copy(
                    acc.at[pl.ds(16 * ROWS_T, N - 16 * ROWS_T)],
                    out_hbm.at[pl.ds(16 * ROWS_T, N - 16 * ROWS_T), pl.ds(col, 128)])

            plsc.subcore_barrier()

    return scatter_kernel(emb2, dst, x2)


def kernel(x, x_edge, edge_distance, edge_index, wigner_inv,
           W1, b1, ln_g, ln_b, W2, b2, to_m, out_idx):
    emb = _compute_emb(x_edge, edge_distance, wigner_inv,
                       W1, b1, ln_g, ln_b, W2, b2)
    out2 = _sc_scatter(emb, edge_index[1], x.reshape(N, FULL * C))
    return out2.reshape(N, FULL, C)
